# baseline (speedup 1.0000x reference)
import functools

import jax
import jax.numpy as jnp
from jax import lax
from jax.experimental import pallas as pl
from jax.experimental.pallas import tpu as pltpu
from jax.experimental.pallas import tpu_sc as plsc

_VOCAB = 100000
_D = 128
_NL = 16
_B = 1024
_S = 200

_NC, _NS = 2, 16
_NW = _NC * _NS
_SPW = _B // _NW
_IPW = _SPW * _S
_CHUNK = 128
_NCHUNK = _IPW // _CHUNK

_PACK = 8
_PROWS = _VOCAB // _PACK
_PROWS_PAD = 12800
_NBLK = 20
_BRO = _PROWS_PAD // _NBLK
_LASTN = _PROWS - (_NBLK - 1) * _BRO
_NBUF = 6


def _proj_body(t_hbm, wb_ref, o_hbm, tbuf, obuf, insem, outsem):
    def in_copy(i, b, u):
        n = _LASTN if i == _NBLK - 1 else _BRO
        return pltpu.make_async_copy(
            t_hbm.at[pl.ds(u * _PROWS + i * _BRO, n), :],
            tbuf.at[b, u, pl.ds(0, n), :], insem.at[b])

    def out_copy(i, b):
        return pltpu.make_async_copy(
            obuf.at[b], o_hbm.at[pl.ds(i * _BRO, _BRO), :], outsem.at[b])

    def start_in(i, b):
        for u in range(_PACK):
            in_copy(i, b, u).start()

    def wait_in(i, b):
        for u in range(_PACK):
            in_copy(i, b, u).wait()

    for i in range(_NBUF):
        start_in(i, i)
    for i in range(_NBLK):
        b = i % _NBUF
        if i >= 1 and i - 1 + _NBUF < _NBLK:
            start_in(i - 1 + _NBUF, (i - 1) % _NBUF)
        wait_in(i, b)
        if i >= _NBUF:
            out_copy(i - _NBUF, b).wait()
        acc = lax.dot_general(
            tbuf[b, 0].astype(jnp.bfloat16), wb_ref[0],
            (((1,), (0,)), ((), ())),
            preferred_element_type=jnp.float32)
        for u in range(1, _PACK):
            acc = acc + lax.dot_general(
                tbuf[b, u].astype(jnp.bfloat16), wb_ref[u],
                (((1,), (0,)), ((), ())),
                preferred_element_type=jnp.float32)
        obuf[b] = acc
        out_copy(i, b).start()
    for i in range(_NBLK - _NBUF, _NBLK):
        out_copy(i, i % _NBUF).wait()


def _project(table, WB):
    return pl.pallas_call(
        _proj_body,
        in_specs=[
            pl.BlockSpec(memory_space=pl.ANY),
            pl.BlockSpec((_PACK, _D, _D), lambda: (0, 0, 0)),
        ],
        out_specs=pl.BlockSpec(memory_space=pl.ANY),
        out_shape=jax.ShapeDtypeStruct((_PROWS_PAD, _PACK * _NL), jnp.float32),
        scratch_shapes=[
            pltpu.VMEM((_NBUF, _PACK, _BRO, _D), jnp.float32),
            pltpu.VMEM((_NBUF, _BRO, _PACK * _NL), jnp.float32),
            pltpu.SemaphoreType.DMA((_NBUF,)),
            pltpu.SemaphoreType.DMA((_NBUF,)),
        ],
    )(table, WB)


_HCHUNK = _NCHUNK // 2
_HROWS = _HCHUNK * _CHUNK
_HSAMP = _SPW // 2


def _sc_body(p_hbm, ids_hbm, b_hbm, out_hbm, ids_v, rows_v, out_v, b_v,
             sem_a, sem_b):
    wid = lax.axis_index("s") * _NC + lax.axis_index("c")
    base = wid * _IPW
    pltpu.sync_copy(ids_hbm.at[pl.ds(base, _IPW)], ids_v)
    pltpu.sync_copy(b_hbm, b_v)

    def make_fire(sem):
        def fire(c, carry):
            pltpu.async_copy(
                p_hbm.at[ids_v.at[pl.ds(c * _CHUNK, _CHUNK)]],
                rows_v.at[pl.ds(c * _CHUNK, _CHUNK)],
                sem,
            )
            return carry

        return fire

    lax.fori_loop(0, _HCHUNK, make_fire(sem_a), 0)
    lax.fori_loop(_HCHUNK, _NCHUNK, make_fire(sem_b), 0)

    bvec = b_v[...]

    def per_sample(s, carry):
        rbase = s * _S

        def add4(j, accs):
            a0, a1, a2, a3 = accs
            r = rbase + j * 4
            return (a0 + rows_v[r], a1 + rows_v[r + 1],
                    a2 + rows_v[r + 2], a3 + rows_v[r + 3])

        z = jnp.zeros((_NL,), jnp.float32)
        a0, a1, a2, a3 = lax.fori_loop(0, _S // 4, add4, (z, z, z, z))
        out_v[s] = ((a0 + a1) + (a2 + a3)) * (1.0 / _S) + bvec
        return carry

    pltpu.make_async_copy(
        p_hbm.at[pl.ds(0, _HROWS)], rows_v.at[pl.ds(0, _HROWS)],
        sem_a).wait()
    lax.fori_loop(0, _HSAMP, per_sample, 0)
    pltpu.make_async_copy(
        p_hbm.at[pl.ds(0, _HROWS)], rows_v.at[pl.ds(_HROWS, _HROWS)],
        sem_b).wait()
    lax.fori_loop(_HSAMP, _SPW, per_sample, 0)
    pltpu.sync_copy(out_v, out_hbm.at[pl.ds(wid * _SPW, _SPW)])


@functools.partial(
    pl.kernel,
    out_type=jax.ShapeDtypeStruct((_B, _NL), jnp.float32),
    mesh=plsc.VectorSubcoreMesh(core_axis_name="c", subcore_axis_name="s"),
    compiler_params=pltpu.CompilerParams(use_tc_tiling_on_sc=False),
    scratch_types=[
        pltpu.VMEM((_IPW,), jnp.int32),
        pltpu.VMEM((_IPW, _NL), jnp.float32),
        pltpu.VMEM((_SPW, _NL), jnp.float32),
        pltpu.VMEM((_NL,), jnp.float32),
        pltpu.SemaphoreType.DMA,
        pltpu.SemaphoreType.DMA,
    ],
)
def _sc_pool(p_hbm, ids_hbm, b_hbm, out_hbm, ids_v, rows_v, out_v, b_v,
             sem_a, sem_b):
    _sc_body(p_hbm, ids_hbm, b_hbm, out_hbm, ids_v, rows_v, out_v, b_v,
             sem_a, sem_b)


def kernel(input_ids, table, W, b):
    eye = jnp.eye(_PACK, dtype=jnp.float32)
    WB = jnp.einsum("uv,ld->udvl", eye, W).reshape(
        _PACK, _D, _PACK * _NL).astype(jnp.bfloat16)
    P = _project(table, WB).reshape(_PROWS_PAD * _PACK, _NL)
    ids_flat = input_ids.reshape(-1).astype(jnp.int32)
    ids_g = 8 * ids_flat - 99999 * (ids_flat // _PROWS)
    return _sc_pool(P, ids_g, b)


# speedup vs baseline: 1.0183x; 1.0183x over previous
import functools

import jax
import jax.numpy as jnp
from jax import lax
from jax.experimental import pallas as pl
from jax.experimental.pallas import tpu as pltpu
from jax.experimental.pallas import tpu_sc as plsc

_VOCAB = 100000
_D = 128
_NL = 16
_B = 1024
_S = 200

_NC, _NS = 2, 16
_NW = _NC * _NS
_SPW = _B // _NW
_IPW = _SPW * _S
_CHUNK = 128
_NCHUNK = _IPW // _CHUNK

_PACK = 8
_PROWS = _VOCAB // _PACK
_PROWS_PAD = 12800
_NBLK = 20
_BRO = _PROWS_PAD // _NBLK
_LASTN = _PROWS - (_NBLK - 1) * _BRO
_NBUF = 4


def _proj_body(t_hbm, wb_ref, o_hbm, tbuf, obuf, insem, outsem):
    def in_copy(i, b, u):
        n = _LASTN if i == _NBLK - 1 else _BRO
        return pltpu.make_async_copy(
            t_hbm.at[pl.ds(u * _PROWS + i * _BRO, n), :],
            tbuf.at[b, u, pl.ds(0, n), :], insem.at[b])

    def out_copy(i, b):
        return pltpu.make_async_copy(
            obuf.at[b], o_hbm.at[pl.ds(i * _BRO, _BRO), :], outsem.at[b])

    def start_in(i, b):
        for u in range(_PACK):
            in_copy(i, b, u).start()

    def wait_in(i, b):
        for u in range(_PACK):
            in_copy(i, b, u).wait()

    for i in range(_NBUF):
        start_in(i, i)
    for i in range(_NBLK):
        b = i % _NBUF
        if i >= 1 and i - 1 + _NBUF < _NBLK:
            start_in(i - 1 + _NBUF, (i - 1) % _NBUF)
        wait_in(i, b)
        if i >= _NBUF:
            out_copy(i - _NBUF, b).wait()
        acc = lax.dot_general(
            tbuf[b, 0].astype(jnp.bfloat16), wb_ref[0],
            (((1,), (0,)), ((), ())),
            preferred_element_type=jnp.float32)
        for u in range(1, _PACK):
            acc = acc + lax.dot_general(
                tbuf[b, u].astype(jnp.bfloat16), wb_ref[u],
                (((1,), (0,)), ((), ())),
                preferred_element_type=jnp.float32)
        obuf[b] = acc
        out_copy(i, b).start()
    for i in range(_NBLK - _NBUF, _NBLK):
        out_copy(i, i % _NBUF).wait()


def _project(table, WB):
    return pl.pallas_call(
        _proj_body,
        in_specs=[
            pl.BlockSpec(memory_space=pl.ANY),
            pl.BlockSpec((_PACK, _D, _D), lambda: (0, 0, 0)),
        ],
        out_specs=pl.BlockSpec(memory_space=pl.ANY),
        out_shape=jax.ShapeDtypeStruct((_PROWS_PAD, _PACK * _NL), jnp.float32),
        scratch_shapes=[
            pltpu.VMEM((_NBUF, _PACK, _BRO, _D), jnp.float32),
            pltpu.VMEM((_NBUF, _BRO, _PACK * _NL), jnp.float32),
            pltpu.SemaphoreType.DMA((_NBUF,)),
            pltpu.SemaphoreType.DMA((_NBUF,)),
        ],
    )(table, WB)


_HCHUNK = _NCHUNK // 2
_HROWS = _HCHUNK * _CHUNK
_HSAMP = _SPW // 2


def _sc_body(p_hbm, ids_hbm, b_hbm, out_hbm, ids_v, rows_v, out_v, b_v,
             sem_a, sem_b):
    wid = lax.axis_index("s") * _NC + lax.axis_index("c")
    base = wid * _IPW
    pltpu.sync_copy(ids_hbm.at[pl.ds(base, _IPW)], ids_v)
    pltpu.sync_copy(b_hbm, b_v)

    def make_fire(sem):
        def fire(c, carry):
            pltpu.async_copy(
                p_hbm.at[ids_v.at[pl.ds(c * _CHUNK, _CHUNK)]],
                rows_v.at[pl.ds(c * _CHUNK, _CHUNK)],
                sem,
            )
            return carry

        return fire

    lax.fori_loop(0, _HCHUNK, make_fire(sem_a), 0)
    lax.fori_loop(_HCHUNK, _NCHUNK, make_fire(sem_b), 0)

    bvec = b_v[...]

    def per_sample(s, carry):
        rbase = s * _S

        def add4(j, accs):
            a0, a1, a2, a3 = accs
            r = rbase + j * 4
            return (a0 + rows_v[r], a1 + rows_v[r + 1],
                    a2 + rows_v[r + 2], a3 + rows_v[r + 3])

        z = jnp.zeros((_NL,), jnp.float32)
        a0, a1, a2, a3 = lax.fori_loop(0, _S // 4, add4, (z, z, z, z))
        out_v[s] = ((a0 + a1) + (a2 + a3)) * (1.0 / _S) + bvec
        return carry

    pltpu.make_async_copy(
        p_hbm.at[pl.ds(0, _HROWS)], rows_v.at[pl.ds(0, _HROWS)],
        sem_a).wait()
    lax.fori_loop(0, _HSAMP, per_sample, 0)
    pltpu.make_async_copy(
        p_hbm.at[pl.ds(0, _HROWS)], rows_v.at[pl.ds(_HROWS, _HROWS)],
        sem_b).wait()
    lax.fori_loop(_HSAMP, _SPW, per_sample, 0)
    pltpu.sync_copy(out_v, out_hbm.at[pl.ds(wid * _SPW, _SPW)])


@functools.partial(
    pl.kernel,
    out_type=jax.ShapeDtypeStruct((_B, _NL), jnp.float32),
    mesh=plsc.VectorSubcoreMesh(core_axis_name="c", subcore_axis_name="s"),
    compiler_params=pltpu.CompilerParams(use_tc_tiling_on_sc=False),
    scratch_types=[
        pltpu.VMEM((_IPW,), jnp.int32),
        pltpu.VMEM((_IPW, _NL), jnp.float32),
        pltpu.VMEM((_SPW, _NL), jnp.float32),
        pltpu.VMEM((_NL,), jnp.float32),
        pltpu.SemaphoreType.DMA,
        pltpu.SemaphoreType.DMA,
    ],
)
def _sc_pool(p_hbm, ids_hbm, b_hbm, out_hbm, ids_v, rows_v, out_v, b_v,
             sem_a, sem_b):
    _sc_body(p_hbm, ids_hbm, b_hbm, out_hbm, ids_v, rows_v, out_v, b_v,
             sem_a, sem_b)


def kernel(input_ids, table, W, b):
    eye = jnp.eye(_PACK, dtype=jnp.float32)
    WB = jnp.einsum("uv,ld->udvl", eye, W).reshape(
        _PACK, _D, _PACK * _NL).astype(jnp.bfloat16)
    P = _project(table, WB).reshape(_PROWS_PAD * _PACK, _NL)
    ids_flat = input_ids.reshape(-1).astype(jnp.int32)
    ids_g = 8 * ids_flat - 99999 * (ids_flat // _PROWS)
    return _sc_pool(P, ids_g, b)


# speedup vs baseline: 1.0974x; 1.0777x over previous
import functools

import jax
import jax.numpy as jnp
from jax import lax
from jax.experimental import pallas as pl
from jax.experimental.pallas import tpu as pltpu
from jax.experimental.pallas import tpu_sc as plsc

_VOCAB = 100000
_D = 128
_NL = 16
_B = 1024
_S = 200

_NC, _NS = 2, 16
_NW = _NC * _NS
_SPW = _B // _NW
_IPW = _SPW * _S
_CHUNK = 128
_NCHUNK = _IPW // _CHUNK

_PACK = 8
_PROWS = _VOCAB // _PACK
_PROWS_PAD = 12800
_NBLK = 20
_BRO = _PROWS_PAD // _NBLK
_LASTN = _PROWS - (_NBLK - 1) * _BRO
_NBUF = 4
_IDROWS = _B * _S // _D


def _proj_body(t_hbm, w_ref, ids_hbm, o_hbm, idsout_hbm,
               tbuf, obuf, wbbuf, idbuf, insem, outsem, idsem, id2sem):
    def in_copy(i, b, u):
        n = _LASTN if i == _NBLK - 1 else _BRO
        return pltpu.make_async_copy(
            t_hbm.at[pl.ds(u * _PROWS + i * _BRO, n), :],
            tbuf.at[b, u, pl.ds(0, n), :], insem.at[b])

    def out_copy(i, b):
        return pltpu.make_async_copy(
            obuf.at[b], o_hbm.at[pl.ds(i * _BRO, _BRO), :], outsem.at[b])

    def start_in(i, b):
        for u in range(_PACK):
            in_copy(i, b, u).start()

    def wait_in(i, b):
        for u in range(_PACK):
            in_copy(i, b, u).wait()

    for i in range(_NBUF):
        start_in(i, i)
    pltpu.make_async_copy(ids_hbm, idbuf, idsem).start()

    w_t = jnp.transpose(w_ref[...], (1, 0))
    wrep = jnp.concatenate([w_t] * _PACK, axis=1)
    wrep_bf = wrep.astype(jnp.bfloat16)
    colgrp = lax.broadcasted_iota(jnp.int32, (_D, _D), 1) // _NL
    zero_bf = jnp.zeros((_D, _D), jnp.bfloat16)
    for u in range(_PACK):
        wbbuf[u] = jnp.where(colgrp == u, wrep_bf, zero_bf)

    pltpu.make_async_copy(ids_hbm, idbuf, idsem).wait()

    def remap(k, carry):
        v = idbuf[pl.ds(k * 64, 64), :]
        idbuf[pl.ds(k * 64, 64), :] = v * _PACK - (v // _PROWS) * 99999
        return carry

    lax.fori_loop(0, _IDROWS // 64, remap, 0)
    pltpu.make_async_copy(idbuf, idsout_hbm, id2sem).start()

    for i in range(_NBLK):
        b = i % _NBUF
        if i >= 1 and i - 1 + _NBUF < _NBLK:
            start_in(i - 1 + _NBUF, (i - 1) % _NBUF)
        wait_in(i, b)
        if i >= _NBUF:
            out_copy(i - _NBUF, b).wait()
        acc = lax.dot_general(
            tbuf[b, 0].astype(jnp.bfloat16), wbbuf[0],
            (((1,), (0,)), ((), ())),
            preferred_element_type=jnp.float32)
        for u in range(1, _PACK):
            acc = acc + lax.dot_general(
                tbuf[b, u].astype(jnp.bfloat16), wbbuf[u],
                (((1,), (0,)), ((), ())),
                preferred_element_type=jnp.float32)
        obuf[b] = acc
        out_copy(i, b).start()
    for i in range(_NBLK - _NBUF, _NBLK):
        out_copy(i, i % _NBUF).wait()
    pltpu.make_async_copy(idbuf, idsout_hbm, id2sem).wait()


def _project(table, W, ids2d):
    return pl.pallas_call(
        _proj_body,
        in_specs=[
            pl.BlockSpec(memory_space=pl.ANY),
            pl.BlockSpec((_NL, _D), lambda: (0, 0)),
            pl.BlockSpec(memory_space=pl.ANY),
        ],
        out_specs=[
            pl.BlockSpec(memory_space=pl.ANY),
            pl.BlockSpec(memory_space=pl.ANY),
        ],
        out_shape=[
            jax.ShapeDtypeStruct((_PROWS_PAD, _PACK * _NL), jnp.float32),
            jax.ShapeDtypeStruct((_IDROWS, _D), jnp.int32),
        ],
        scratch_shapes=[
            pltpu.VMEM((_NBUF, _PACK, _BRO, _D), jnp.float32),
            pltpu.VMEM((_NBUF, _BRO, _PACK * _NL), jnp.float32),
            pltpu.VMEM((_PACK, _D, _D), jnp.bfloat16),
            pltpu.VMEM((_IDROWS, _D), jnp.int32),
            pltpu.SemaphoreType.DMA((_NBUF,)),
            pltpu.SemaphoreType.DMA((_NBUF,)),
            pltpu.SemaphoreType.DMA,
            pltpu.SemaphoreType.DMA,
        ],
    )(table, W, ids2d)


_HCHUNK = _NCHUNK // 2
_HROWS = _HCHUNK * _CHUNK
_HSAMP = _SPW // 2


def _sc_body(p_hbm, ids_hbm, b_hbm, out_hbm, ids_v, rows_v, out_v, b_v,
             sem_a, sem_b):
    wid = lax.axis_index("s") * _NC + lax.axis_index("c")
    pltpu.sync_copy(ids_hbm.at[pl.ds(wid * _NCHUNK, _NCHUNK), :], ids_v)
    pltpu.sync_copy(b_hbm, b_v)

    def make_fire(sem):
        def fire(c, carry):
            pltpu.async_copy(
                p_hbm.at[ids_v.at[c]],
                rows_v.at[pl.ds(c * _CHUNK, _CHUNK)],
                sem,
            )
            return carry

        return fire

    lax.fori_loop(0, _HCHUNK, make_fire(sem_a), 0)
    lax.fori_loop(_HCHUNK, _NCHUNK, make_fire(sem_b), 0)

    bvec = b_v[...]

    def per_sample(s, carry):
        rbase = s * _S

        def add4(j, accs):
            a0, a1, a2, a3 = accs
            r = rbase + j * 4
            return (a0 + rows_v[r], a1 + rows_v[r + 1],
                    a2 + rows_v[r + 2], a3 + rows_v[r + 3])

        z = jnp.zeros((_NL,), jnp.float32)
        a0, a1, a2, a3 = lax.fori_loop(0, _S // 4, add4, (z, z, z, z))
        out_v[s] = ((a0 + a1) + (a2 + a3)) * (1.0 / _S) + bvec
        return carry

    pltpu.make_async_copy(
        p_hbm.at[pl.ds(0, _HROWS)], rows_v.at[pl.ds(0, _HROWS)],
        sem_a).wait()
    lax.fori_loop(0, _HSAMP, per_sample, 0)
    pltpu.make_async_copy(
        p_hbm.at[pl.ds(0, _HROWS)], rows_v.at[pl.ds(_HROWS, _HROWS)],
        sem_b).wait()
    lax.fori_loop(_HSAMP, _SPW, per_sample, 0)
    pltpu.sync_copy(out_v, out_hbm.at[pl.ds(wid * _SPW, _SPW)])


@functools.partial(
    pl.kernel,
    out_type=jax.ShapeDtypeStruct((_B, _NL), jnp.float32),
    mesh=plsc.VectorSubcoreMesh(core_axis_name="c", subcore_axis_name="s"),
    compiler_params=pltpu.CompilerParams(use_tc_tiling_on_sc=False),
    scratch_types=[
        pltpu.VMEM((_NCHUNK, _CHUNK), jnp.int32),
        pltpu.VMEM((_IPW, _NL), jnp.float32),
        pltpu.VMEM((_SPW, _NL), jnp.float32),
        pltpu.VMEM((_NL,), jnp.float32),
        pltpu.SemaphoreType.DMA,
        pltpu.SemaphoreType.DMA,
    ],
)
def _sc_pool(p_hbm, ids_hbm, b_hbm, out_hbm, ids_v, rows_v, out_v, b_v,
             sem_a, sem_b):
    _sc_body(p_hbm, ids_hbm, b_hbm, out_hbm, ids_v, rows_v, out_v, b_v,
             sem_a, sem_b)


def kernel(input_ids, table, W, b):
    ids2d = input_ids.astype(jnp.int32).reshape(_IDROWS, _D)
    P, ids_g = _project(table, W, ids2d)
    return _sc_pool(P.reshape(_PROWS_PAD * _PACK, _NL), ids_g, b)
